# Initial kernel scaffold; baseline (speedup 1.0000x reference)
import functools

import jax
import jax.numpy as jnp
from jax import lax
from jax.experimental import pallas as pl
from jax.experimental.pallas import tpu as pltpu
from jax.experimental.pallas import tpu_sc as plsc

_C = 1.0
_NC = 2
_NS = 16
_L = 16
_NW = _NC * _NS
_BLK = 80


def _node_table_body(x_ref, wp_ref, b_ref, ta_ref, tb_ref):
    x = x_ref[...]
    x0 = x[:, 0:1]
    total = jnp.sum(x * x, axis=1, keepdims=True)
    nsq = jnp.maximum(total - x0 * x0, 0.0)
    norm = jnp.maximum(jnp.sqrt(nsq), 1e-8)
    x0c = jnp.maximum(x0, 1.0 + 1e-6)
    dist = jnp.log(x0c + jnp.sqrt((x0c - 1.0) * (x0c + 1.0)))
    s = dist / norm
    mu = jnp.dot(x, wp_ref[...], preferred_element_type=jnp.float32) * s
    mu = mu + b_ref[0:1, :]
    mn = jnp.maximum(jnp.sqrt(jnp.sum(mu * mu, axis=1, keepdims=True)), 1e-8)
    e = jnp.exp(mn)
    ei = 1.0 / e
    ch = 0.5 * (e + ei)
    sh = 0.5 * (e - ei)
    tail = (sh / mn) * mu
    used = tail[:, : x.shape[1] - 1]
    ta_ref[...] = jnp.concatenate([-ch, used], axis=1)
    tb_ref[...] = jnp.concatenate([ch, used], axis=1)


def _make_node_tables(x, wp, b8, bn):
    n, d_in = x.shape
    grid = n // bn
    return pl.pallas_call(
        _node_table_body,
        grid=(grid,),
        in_specs=[
            pl.BlockSpec((bn, d_in), lambda i: (i, 0)),
            pl.BlockSpec((d_in, d_in), lambda i: (0, 0)),
            pl.BlockSpec((8, d_in), lambda i: (0, 0)),
        ],
        out_specs=[
            pl.BlockSpec((bn, d_in), lambda i: (i, 0)),
            pl.BlockSpec((bn, d_in), lambda i: (i, 0)),
        ],
        out_shape=[
            jax.ShapeDtypeStruct((n, d_in), jnp.float32),
            jax.ShapeDtypeStruct((n, d_in), jnp.float32),
        ],
    )(x, wp, b8)


def _make_edge_kernel(e_total, d_in):
    mesh = plsc.VectorSubcoreMesh(
        core_axis_name="c", subcore_axis_name="s", num_cores=_NC
    )
    ept = e_total // _NW
    nblk = ept // _BLK
    groups = _BLK // _L

    @functools.partial(
        pl.kernel,
        mesh=mesh,
        out_type=jax.ShapeDtypeStruct((e_total,), jnp.float32),
        scratch_types=[
            pltpu.VMEM((_BLK,), jnp.int32),
            pltpu.VMEM((_BLK,), jnp.int32),
            pltpu.VMEM((_BLK, d_in), jnp.float32),
            pltpu.VMEM((_BLK, d_in), jnp.float32),
            pltpu.VMEM((_BLK,), jnp.float32),
            pltpu.SemaphoreType.DMA,
            pltpu.SemaphoreType.DMA,
        ],
    )
    def edge_kernel(ta_hbm, tb_hbm, src_hbm, dst_hbm, out_hbm,
                    idx_s, idx_d, rows_s, rows_d, res_v, sem_a, sem_b):
        wid = lax.axis_index("s") * _NC + lax.axis_index("c")
        tile_base = wid * ept
        iota = lax.iota(jnp.int32, _L)

        def block_body(k, carry):
            base = tile_base + k * _BLK
            pltpu.sync_copy(src_hbm.at[pl.ds(base, _BLK)], idx_s)
            pltpu.sync_copy(dst_hbm.at[pl.ds(base, _BLK)], idx_d)
            cp_a = pltpu.async_copy(ta_hbm.at[idx_s], rows_s, sem_a)
            cp_b = pltpu.async_copy(tb_hbm.at[idx_d], rows_d, sem_b)
            cp_a.wait()
            cp_b.wait()
            for g in range(groups):
                rvec = iota + (g * _L)

                def dim_body(d, acc):
                    col = jnp.full((_L,), d, jnp.int32)
                    a = plsc.load_gather(rows_s, [rvec, col])
                    bb = plsc.load_gather(rows_d, [rvec, col])
                    return acc + a * bb

                acc = lax.fori_loop(0, d_in, dim_body,
                                    jnp.zeros((_L,), jnp.float32), unroll=4)
                t = -_C - acc
                r = jnp.minimum(jnp.maximum(t, 1e-10), 1.0)
                res_v[pl.ds(g * _L, _L)] = jnp.exp(-r)
            pltpu.sync_copy(res_v, out_hbm.at[pl.ds(base, _BLK)])
            return carry

        lax.fori_loop(0, nblk, block_body, 0)

    return edge_kernel


def kernel(x, edge_index, W, b):
    n, d_in = x.shape
    e = edge_index.shape[1]
    x = x.astype(jnp.float32)
    wp = jnp.concatenate(
        [jnp.zeros((1, d_in), jnp.float32), W.astype(jnp.float32)], axis=0
    )
    b8 = jnp.broadcast_to(b.astype(jnp.float32), (8, d_in))

    bn = 400
    n_pad = ((n + bn - 1) // bn) * bn
    xp = x if n_pad == n else jnp.pad(x, ((0, n_pad - n), (0, 0)))
    ta, tb = _make_node_tables(xp, wp, b8, bn)
    ta = ta[:n]
    tb = tb[:n]

    src = edge_index[0].astype(jnp.int32)
    dst = edge_index[1].astype(jnp.int32)
    chunk = _NW * _BLK
    e_pad = ((e + chunk - 1) // chunk) * chunk
    if e_pad != e:
        src = jnp.pad(src, (0, e_pad - e))
        dst = jnp.pad(dst, (0, e_pad - e))

    res = _make_edge_kernel(e_pad, d_in)(ta, tb, src, dst)
    if e_pad != e:
        res = res[:e]
    return (edge_index, res, (n, n))


# speedup vs baseline: 1.5190x; 1.5190x over previous
import functools

import jax
import jax.numpy as jnp
from jax import lax
from jax.experimental import pallas as pl
from jax.experimental.pallas import tpu as pltpu
from jax.experimental.pallas import tpu_sc as plsc

_C = 1.0
_NC = 2
_NS = 16
_L = 16
_NW = _NC * _NS
_BLK = 80


def _node_table_body(x_ref, wp_ref, b_ref, tab_ref):
    x = x_ref[...]
    x0 = x[:, 0:1]
    total = jnp.sum(x * x, axis=1, keepdims=True)
    nsq = jnp.maximum(total - x0 * x0, 0.0)
    norm = jnp.maximum(jnp.sqrt(nsq), 1e-8)
    x0c = jnp.maximum(x0, 1.0 + 1e-6)
    dist = jnp.log(x0c + jnp.sqrt((x0c - 1.0) * (x0c + 1.0)))
    s = dist / norm
    mu = jnp.dot(x, wp_ref[...], preferred_element_type=jnp.float32) * s
    mu = mu + b_ref[0:1, :]
    mn = jnp.maximum(jnp.sqrt(jnp.sum(mu * mu, axis=1, keepdims=True)), 1e-8)
    e = jnp.exp(mn)
    ei = 1.0 / e
    ch = 0.5 * (e + ei)
    sh = 0.5 * (e - ei)
    tail = (sh / mn) * mu
    used = tail[:, : x.shape[1] - 1]
    tab_ref[0] = jnp.concatenate([-ch, used], axis=1)
    tab_ref[1] = jnp.concatenate([ch, used], axis=1)


def _make_node_table(x, wp, b8, bn):
    n, d_in = x.shape
    grid = n // bn
    return pl.pallas_call(
        _node_table_body,
        grid=(grid,),
        in_specs=[
            pl.BlockSpec((bn, d_in), lambda i: (i, 0)),
            pl.BlockSpec((d_in, d_in), lambda i: (0, 0)),
            pl.BlockSpec((8, d_in), lambda i: (0, 0)),
        ],
        out_specs=pl.BlockSpec((2, bn, d_in), lambda i: (0, i, 0)),
        out_shape=jax.ShapeDtypeStruct((2, n, d_in), jnp.float32),
    )(x, wp, b8)


def _make_edge_kernel(e_total, d_in):
    mesh = plsc.VectorSubcoreMesh(
        core_axis_name="c", subcore_axis_name="s", num_cores=_NC
    )
    ept = e_total // _NW
    nblk = ept // _BLK
    groups = _BLK // _L
    rows_per_blk = 2 * _BLK

    @functools.partial(
        pl.kernel,
        mesh=mesh,
        compiler_params=pltpu.CompilerParams(needs_layout_passes=False),
        out_type=jax.ShapeDtypeStruct((e_total,), jnp.float32),
        scratch_types=[
            pltpu.VMEM((2 * ept,), jnp.int32),
            pltpu.VMEM((rows_per_blk, d_in), jnp.float32),
            pltpu.VMEM((rows_per_blk, d_in), jnp.float32),
            pltpu.VMEM((ept,), jnp.float32),
            pltpu.SemaphoreType.DMA,
            pltpu.SemaphoreType.DMA,
        ],
    )
    def edge_kernel(tab_hbm, idx_hbm, out_hbm,
                    idx_v, rows0, rows1, res_v, sem0, sem1):
        wid = lax.axis_index("s") * _NC + lax.axis_index("c")
        iota = lax.iota(jnp.int32, _L)

        pltpu.sync_copy(idx_hbm.at[pl.ds(wid * 2 * ept, 2 * ept)], idx_v)

        def gather_src(k):
            return tab_hbm.at[idx_v.at[pl.ds(k * rows_per_blk, rows_per_blk)]]

        pltpu.async_copy(gather_src(0), rows0, sem0)

        def do_block(k, buf_cur, sem_cur, buf_nxt, sem_nxt):
            pltpu.make_async_copy(gather_src(k), buf_cur, sem_cur).wait()

            @pl.when(k < nblk - 1)
            def _prefetch():
                pltpu.async_copy(gather_src(k + 1), buf_nxt, sem_nxt)

            def dim_body(d, accs):
                col = jnp.full((_L,), d, jnp.int32)
                out = []
                for g in range(groups):
                    rs = iota + (g * _L)
                    rd = rs + _BLK
                    a = plsc.load_gather(buf_cur, [rs, col])
                    b2 = plsc.load_gather(buf_cur, [rd, col])
                    out.append(accs[g] + a * b2)
                return tuple(out)

            accs = lax.fori_loop(
                0, d_in, dim_body,
                tuple(jnp.zeros((_L,), jnp.float32) for _ in range(groups)),
                unroll=4)
            for g in range(groups):
                t = -_C - accs[g]
                r = jnp.minimum(jnp.maximum(t, 1e-10), 1.0)
                res_v[pl.ds(k * _BLK + g * _L, _L)] = jnp.exp(-r)

        def block_body(k, carry):
            @pl.when(lax.rem(k, 2) == 0)
            def _even():
                do_block(k, rows0, sem0, rows1, sem1)

            @pl.when(lax.rem(k, 2) == 1)
            def _odd():
                do_block(k, rows1, sem1, rows0, sem0)

            return carry

        lax.fori_loop(0, nblk, block_body, 0)
        pltpu.sync_copy(res_v, out_hbm.at[pl.ds(wid * ept, ept)])

    return edge_kernel


def kernel(x, edge_index, W, b):
    n, d_in = x.shape
    e = edge_index.shape[1]
    x = x.astype(jnp.float32)
    wp = jnp.concatenate(
        [jnp.zeros((1, d_in), jnp.float32), W.astype(jnp.float32)], axis=0
    )
    b8 = jnp.broadcast_to(b.astype(jnp.float32), (8, d_in))

    bn = 400
    n_pad = ((n + bn - 1) // bn) * bn
    xp = x if n_pad == n else jnp.pad(x, ((0, n_pad - n), (0, 0)))
    tab = _make_node_table(xp, wp, b8, bn).reshape(2 * n_pad, d_in)

    src = edge_index[0].astype(jnp.int32)
    dst = edge_index[1].astype(jnp.int32)
    chunk = _NW * _BLK
    e_pad = ((e + chunk - 1) // chunk) * chunk
    if e_pad != e:
        src = jnp.pad(src, (0, e_pad - e))
        dst = jnp.pad(dst, (0, e_pad - e))
    nblk = e_pad // chunk
    s3 = src.reshape(_NW, nblk, _BLK)
    d3 = dst.reshape(_NW, nblk, _BLK) + n_pad
    idx_cat = jnp.concatenate([s3, d3], axis=2).reshape(2 * e_pad)

    res = _make_edge_kernel(e_pad, d_in)(tab, idx_cat)
    if e_pad != e:
        res = res[:e]
    return (edge_index, res, (n, n))
